# initial kernel scaffold (unmeasured)
import jax
import jax.numpy as jnp
from jax import lax
from jax.experimental import pallas as pl
from jax.experimental.pallas import tpu as pltpu

B, S, D = 2, 512, 2048
H, Dh, Dr = 16, 128, 32
DC = 128
M = B * S
SCALE = (Dh + Dr) ** -0.5
BF = jnp.bfloat16
F32 = jnp.float32


def _comm_kv_body(x_ref, wdkv_ref, wuk_ref, wuv_ref, wkr_ref,
                  xb_ref, k_ref, v_ref, kr_ref,
                  wdkv_o, wuk_o, wuv_o, send_sems, recv_sems):
    my_x = lax.axis_index("x")
    my_y = lax.axis_index("y")
    my_z = lax.axis_index("z")
    nbr = (1 - my_x, my_y, my_z)

    barrier = pltpu.get_barrier_semaphore()
    pl.semaphore_signal(barrier, inc=1, device_id=nbr,
                        device_id_type=pl.DeviceIdType.MESH)
    pl.semaphore_wait(barrier, 1)

    rdmas = []
    for i, (src, dst) in enumerate(
        [(wdkv_ref, wdkv_o), (wuk_ref, wuk_o), (wuv_ref, wuv_o)]
    ):
        rdma = pltpu.make_async_remote_copy(
            src_ref=src, dst_ref=dst,
            send_sem=send_sems.at[i], recv_sem=recv_sems.at[i],
            device_id=nbr, device_id_type=pl.DeviceIdType.MESH,
        )
        rdma.start()
        rdmas.append(rdma)

    xb = x_ref[...].reshape(M, D).astype(BF)
    xb_ref[...] = xb
    kr_ref[...] = jnp.dot(
        xb, wkr_ref[...].astype(BF), preferred_element_type=F32
    ).astype(BF)
    c_mine = jnp.dot(
        xb, wdkv_ref[...].astype(BF), preferred_element_type=F32
    ).astype(BF)

    for rdma in rdmas:
        rdma.wait()

    c_other = jnp.dot(
        xb, wdkv_o[...].astype(BF), preferred_element_type=F32
    ).astype(BF)
    k = jnp.dot(c_mine, wuk_ref[...].astype(BF), preferred_element_type=F32)
    k += jnp.dot(c_other, wuk_o[...].astype(BF), preferred_element_type=F32)
    k_ref[...] = k.astype(BF)
    v = jnp.dot(c_mine, wuv_ref[...].astype(BF), preferred_element_type=F32)
    v += jnp.dot(c_other, wuv_o[...].astype(BF), preferred_element_type=F32)
    v_ref[...] = v.astype(BF)


def _qproj_body(xb_ref, wq_ref, wqr_ref, q_ref, qr_ref):
    xb = xb_ref[...]
    q_ref[...] = jnp.dot(
        xb, wq_ref[...].astype(BF), preferred_element_type=F32
    ).astype(BF)
    qr_ref[...] = jnp.dot(
        xb, wqr_ref[...].astype(BF), preferred_element_type=F32
    ).astype(BF)


def _attn_body(q_ref, k_ref, v_ref, qr_ref, kr_ref, o_ref):
    nt = (((1,), (1,)), ((), ()))
    s = lax.dot_general(q_ref[...], k_ref[...], nt, preferred_element_type=F32)
    s += lax.dot_general(qr_ref[...], kr_ref[...], nt, preferred_element_type=F32)
    s *= SCALE
    s -= jnp.max(s, axis=-1, keepdims=True)
    p = jnp.exp(s)
    p /= jnp.sum(p, axis=-1, keepdims=True)
    o_ref[...] = jnp.dot(
        p.astype(BF), v_ref[...], preferred_element_type=F32
    ).astype(BF)


def _oproj_body(o_ref, wo_ref, out_ref):
    out = jnp.dot(o_ref[...], wo_ref[...].astype(BF), preferred_element_type=F32)
    out_ref[...] = out.reshape(B, S, out.shape[-1])


def kernel(x, Wdkv, Wuk, Wuv, Wq, Wqr, Wkr, Wo):
    vmem = pl.BlockSpec(memory_space=pltpu.VMEM)

    xb, K, V, Kr = pl.pallas_call(
        _comm_kv_body,
        out_shape=[
            jax.ShapeDtypeStruct((M, D), BF),
            jax.ShapeDtypeStruct((M, D), BF),
            jax.ShapeDtypeStruct((M, D), BF),
            jax.ShapeDtypeStruct((M, Dr), BF),
        ],
        in_specs=[vmem] * 5,
        out_specs=[vmem] * 4,
        scratch_shapes=[
            pltpu.VMEM((D, DC), F32),
            pltpu.VMEM((DC, D), F32),
            pltpu.VMEM((DC, D), F32),
            pltpu.SemaphoreType.DMA((3,)),
            pltpu.SemaphoreType.DMA((3,)),
        ],
        compiler_params=pltpu.CompilerParams(collective_id=0),
    )(x, Wdkv, Wuk, Wuv, Wkr)

    NJ = 4
    Q, Qr = pl.pallas_call(
        _qproj_body,
        grid=(NJ,),
        in_specs=[
            pl.BlockSpec((M, D), lambda j: (0, 0)),
            pl.BlockSpec((D, D // NJ), lambda j: (0, j)),
            pl.BlockSpec((D, H * Dr // NJ), lambda j: (0, j)),
        ],
        out_specs=[
            pl.BlockSpec((M, D // NJ), lambda j: (0, j)),
            pl.BlockSpec((M, H * Dr // NJ), lambda j: (0, j)),
        ],
        out_shape=[
            jax.ShapeDtypeStruct((M, D), BF),
            jax.ShapeDtypeStruct((M, H * Dr), BF),
        ],
    )(xb, Wq, Wqr)

    O = pl.pallas_call(
        _attn_body,
        grid=(B * H,),
        in_specs=[
            pl.BlockSpec((S, Dh), lambda i: (i // H, i % H)),
            pl.BlockSpec((S, Dh), lambda i: (i // H, i % H)),
            pl.BlockSpec((S, Dh), lambda i: (i // H, i % H)),
            pl.BlockSpec((S, Dr), lambda i: (i // H, i % H)),
            pl.BlockSpec((S, Dr), lambda i: (i // H, 0)),
        ],
        out_specs=pl.BlockSpec((S, Dh), lambda i: (i // H, i % H)),
        out_shape=jax.ShapeDtypeStruct((M, D), BF),
    )(Q, K, V, Qr, Kr)

    out = pl.pallas_call(
        _oproj_body,
        grid=(NJ,),
        in_specs=[
            pl.BlockSpec((M, D), lambda j: (0, 0)),
            pl.BlockSpec((D, D // NJ), lambda j: (0, j)),
        ],
        out_specs=pl.BlockSpec((B, S, D // NJ), lambda j: (0, 0, j)),
        out_shape=jax.ShapeDtypeStruct((B, S, D), F32),
    )(O, Wo)
    return out


# baseline (device time: 124156 ns/iter reference)
import jax
import jax.numpy as jnp
from jax import lax
from jax.experimental import pallas as pl
from jax.experimental.pallas import tpu as pltpu

B, S, D = 2, 512, 2048
H, Dh, Dr = 16, 128, 32
DC = 128
M = B * S
SCALE = (Dh + Dr) ** -0.5
BF = jnp.bfloat16
F32 = jnp.float32


def _comm_kv_body(x_ref, wdkv_ref, wuk_ref, wuv_ref, wkr_ref,
                  xb_ref, k_ref, v_ref, kr_ref,
                  wdkv_o, wuk_o, wuv_o, send_sems, recv_sems):
    my_x = lax.axis_index("x")
    my_y = lax.axis_index("y")
    my_z = lax.axis_index("z")
    nbr = (1 - my_x, my_y, my_z)

    barrier = pltpu.get_barrier_semaphore()
    pl.semaphore_signal(barrier, inc=1, device_id=nbr,
                        device_id_type=pl.DeviceIdType.MESH)
    pl.semaphore_wait(barrier, 1)

    rdmas = []
    for i, (src, dst) in enumerate(
        [(wdkv_ref, wdkv_o), (wuk_ref, wuk_o), (wuv_ref, wuv_o)]
    ):
        rdma = pltpu.make_async_remote_copy(
            src_ref=src, dst_ref=dst,
            send_sem=send_sems.at[i], recv_sem=recv_sems.at[i],
            device_id=nbr, device_id_type=pl.DeviceIdType.MESH,
        )
        rdma.start()
        rdmas.append(rdma)

    xb = x_ref[...].reshape(M, D).astype(BF)
    xb_ref[...] = xb
    kr_ref[...] = jnp.dot(
        xb, wkr_ref[...].astype(BF), preferred_element_type=F32
    ).astype(BF)
    c_mine = jnp.dot(
        xb, wdkv_ref[...].astype(BF), preferred_element_type=F32
    ).astype(BF)

    for rdma in rdmas:
        rdma.wait()

    c_other = jnp.dot(
        xb, wdkv_o[...].astype(BF), preferred_element_type=F32
    ).astype(BF)
    k = jnp.dot(c_mine, wuk_ref[...].astype(BF), preferred_element_type=F32)
    k += jnp.dot(c_other, wuk_o[...].astype(BF), preferred_element_type=F32)
    k_ref[...] = k.astype(BF)
    v = jnp.dot(c_mine, wuv_ref[...].astype(BF), preferred_element_type=F32)
    v += jnp.dot(c_other, wuv_o[...].astype(BF), preferred_element_type=F32)
    v_ref[...] = v.astype(BF)


def _qproj_body(xb_ref, wq_ref, wqr_ref, q_ref, qr_ref):
    xb = xb_ref[...]
    q_ref[...] = jnp.dot(
        xb, wq_ref[...].astype(BF), preferred_element_type=F32
    ).astype(BF)
    qr_ref[...] = jnp.dot(
        xb, wqr_ref[...].astype(BF), preferred_element_type=F32
    ).astype(BF)


def _attn_body(q_ref, k_ref, v_ref, qr_ref, kr_ref, o_ref):
    nt = (((1,), (1,)), ((), ()))
    kr = kr_ref[...]
    for h in range(H):
        q = q_ref[:, h * Dh:(h + 1) * Dh]
        k = k_ref[:, h * Dh:(h + 1) * Dh]
        qr = qr_ref[:, h * Dr:(h + 1) * Dr]
        s = lax.dot_general(q, k, nt, preferred_element_type=F32)
        s += lax.dot_general(qr, kr, nt, preferred_element_type=F32)
        s *= SCALE
        s -= jnp.max(s, axis=-1, keepdims=True)
        p = jnp.exp(s)
        p /= jnp.sum(p, axis=-1, keepdims=True)
        o_ref[:, h * Dh:(h + 1) * Dh] = jnp.dot(
            p.astype(BF), v_ref[:, h * Dh:(h + 1) * Dh],
            preferred_element_type=F32,
        ).astype(BF)


def _oproj_body(o_ref, wo_ref, out_ref):
    out = jnp.dot(o_ref[...], wo_ref[...].astype(BF), preferred_element_type=F32)
    out_ref[...] = out.reshape(B, S, out.shape[-1])


def kernel(x, Wdkv, Wuk, Wuv, Wq, Wqr, Wkr, Wo):
    vmem = pl.BlockSpec(memory_space=pltpu.VMEM)

    xb, K, V, Kr = pl.pallas_call(
        _comm_kv_body,
        out_shape=[
            jax.ShapeDtypeStruct((M, D), BF),
            jax.ShapeDtypeStruct((M, D), BF),
            jax.ShapeDtypeStruct((M, D), BF),
            jax.ShapeDtypeStruct((M, Dr), BF),
        ],
        in_specs=[vmem] * 5,
        out_specs=[vmem] * 4,
        scratch_shapes=[
            pltpu.VMEM((D, DC), F32),
            pltpu.VMEM((DC, D), F32),
            pltpu.VMEM((DC, D), F32),
            pltpu.SemaphoreType.DMA((3,)),
            pltpu.SemaphoreType.DMA((3,)),
        ],
        compiler_params=pltpu.CompilerParams(collective_id=0),
    )(x, Wdkv, Wuk, Wuv, Wkr)

    NJ = 4
    Q, Qr = pl.pallas_call(
        _qproj_body,
        grid=(NJ,),
        in_specs=[
            pl.BlockSpec((M, D), lambda j: (0, 0)),
            pl.BlockSpec((D, D // NJ), lambda j: (0, j)),
            pl.BlockSpec((D, H * Dr // NJ), lambda j: (0, j)),
        ],
        out_specs=[
            pl.BlockSpec((M, D // NJ), lambda j: (0, j)),
            pl.BlockSpec((M, H * Dr // NJ), lambda j: (0, j)),
        ],
        out_shape=[
            jax.ShapeDtypeStruct((M, D), BF),
            jax.ShapeDtypeStruct((M, H * Dr), BF),
        ],
    )(xb, Wq, Wqr)

    O = pl.pallas_call(
        _attn_body,
        grid=(B,),
        in_specs=[
            pl.BlockSpec((S, D), lambda b: (b, 0)),
            pl.BlockSpec((S, D), lambda b: (b, 0)),
            pl.BlockSpec((S, D), lambda b: (b, 0)),
            pl.BlockSpec((S, H * Dr), lambda b: (b, 0)),
            pl.BlockSpec((S, Dr), lambda b: (b, 0)),
        ],
        out_specs=pl.BlockSpec((S, D), lambda b: (b, 0)),
        out_shape=jax.ShapeDtypeStruct((M, D), BF),
    )(Q, K, V, Qr, Kr)

    out = pl.pallas_call(
        _oproj_body,
        grid=(NJ,),
        in_specs=[
            pl.BlockSpec((M, D), lambda j: (0, 0)),
            pl.BlockSpec((D, D // NJ), lambda j: (0, j)),
        ],
        out_specs=pl.BlockSpec((B, S, D // NJ), lambda j: (0, 0, j)),
        out_shape=jax.ShapeDtypeStruct((B, S, D), F32),
    )(O, Wo)
    return out


# device time: 79773 ns/iter; 1.5564x vs baseline; 1.5564x over previous
import jax
import jax.numpy as jnp
from jax import lax
from jax.experimental import pallas as pl
from jax.experimental.pallas import tpu as pltpu

B, S, D = 2, 512, 2048
H, Dh, Dr = 16, 128, 32
DC = 128
M = B * S
SCALE = (Dh + Dr) ** -0.5
BF = jnp.bfloat16
F32 = jnp.float32
NJ = 4


def _proj_comm_body(x_ref, wdkv_ref, wuk_ref, wuv_ref, wkr_ref, wq_ref, wqr_ref,
                    q_ref, qr_ref, k_ref, v_ref, kr_ref,
                    xb_s, wdkv_s, wuk_s, wuv_s, wdkv_r, wuk_r, wuv_r,
                    send_sems, recv_sems):
    j = pl.program_id(0)
    my_x = lax.axis_index("x")
    my_y = lax.axis_index("y")
    my_z = lax.axis_index("z")
    nbr = (1 - my_x, my_y, my_z)

    pairs = [(wdkv_s, wdkv_r), (wuk_s, wuk_r), (wuv_s, wuv_r)]

    def mk(i, src, dst):
        return pltpu.make_async_remote_copy(
            src_ref=src, dst_ref=dst,
            send_sem=send_sems.at[i], recv_sem=recv_sems.at[i],
            device_id=nbr, device_id_type=pl.DeviceIdType.MESH,
        )

    @pl.when(j == 0)
    def _():
        xb_s[...] = x_ref[...].reshape(M, D).astype(BF)
        wdkv_s[...] = wdkv_ref[...].astype(BF)
        wuk_s[...] = wuk_ref[...].astype(BF)
        wuv_s[...] = wuv_ref[...].astype(BF)
        barrier = pltpu.get_barrier_semaphore()
        pl.semaphore_signal(barrier, inc=1, device_id=nbr,
                            device_id_type=pl.DeviceIdType.MESH)
        pl.semaphore_wait(barrier, 1)
        for i, (s, d) in enumerate(pairs):
            mk(i, s, d).start()

    xb = xb_s[...]
    q_ref[...] = jnp.dot(
        xb, wq_ref[...].astype(BF), preferred_element_type=F32
    ).astype(BF)
    qr_ref[...] = jnp.dot(
        xb, wqr_ref[...].astype(BF), preferred_element_type=F32
    ).astype(BF)

    @pl.when(j == NJ - 1)
    def _():
        kr_ref[...] = jnp.dot(
            xb, wkr_ref[...].astype(BF), preferred_element_type=F32
        ).astype(BF)
        for i, (s, d) in enumerate(pairs):
            mk(i, s, d).wait()
        c1 = jnp.dot(xb, wdkv_s[...], preferred_element_type=F32).astype(BF)
        c2 = jnp.dot(xb, wdkv_r[...], preferred_element_type=F32).astype(BF)
        k = jnp.dot(c1, wuk_s[...], preferred_element_type=F32)
        k += jnp.dot(c2, wuk_r[...], preferred_element_type=F32)
        k_ref[...] = k.astype(BF)
        v = jnp.dot(c1, wuv_s[...], preferred_element_type=F32)
        v += jnp.dot(c2, wuv_r[...], preferred_element_type=F32)
        v_ref[...] = v.astype(BF)


def _attn_o_body(q_ref, k_ref, v_ref, qr_ref, kr_ref, wo_ref, out_ref,
                 o_s, wo_bf):
    b = pl.program_id(0)

    @pl.when(b == 0)
    def _():
        wo_bf[...] = wo_ref[...].astype(BF)

    nt = (((1,), (1,)), ((), ()))
    kr = kr_ref[...]
    for h in range(H):
        q = q_ref[:, h * Dh:(h + 1) * Dh]
        k = k_ref[:, h * Dh:(h + 1) * Dh]
        qr = qr_ref[:, h * Dr:(h + 1) * Dr]
        s = lax.dot_general(q, k, nt, preferred_element_type=F32)
        s += lax.dot_general(qr, kr, nt, preferred_element_type=F32)
        p = jnp.exp(s * SCALE)
        p /= jnp.sum(p, axis=-1, keepdims=True)
        o_s[:, h * Dh:(h + 1) * Dh] = jnp.dot(
            p.astype(BF), v_ref[:, h * Dh:(h + 1) * Dh],
            preferred_element_type=F32,
        ).astype(BF)

    out = jnp.dot(o_s[...], wo_bf[...], preferred_element_type=F32)
    out_ref[...] = out.reshape(1, S, D)


def kernel(x, Wdkv, Wuk, Wuv, Wq, Wqr, Wkr, Wo):
    BD = D // NJ
    BR = H * Dr // NJ

    Q, Qr, K, V, Kr = pl.pallas_call(
        _proj_comm_body,
        grid=(NJ,),
        in_specs=[
            pl.BlockSpec((B, S, D), lambda j: (0, 0, 0)),
            pl.BlockSpec((D, DC), lambda j: (0, 0)),
            pl.BlockSpec((DC, D), lambda j: (0, 0)),
            pl.BlockSpec((DC, D), lambda j: (0, 0)),
            pl.BlockSpec((D, Dr), lambda j: (0, 0)),
            pl.BlockSpec((D, BD), lambda j: (0, j)),
            pl.BlockSpec((D, BR), lambda j: (0, j)),
        ],
        out_specs=[
            pl.BlockSpec((M, BD), lambda j: (0, j)),
            pl.BlockSpec((M, BR), lambda j: (0, j)),
            pl.BlockSpec((M, D), lambda j: (0, 0)),
            pl.BlockSpec((M, D), lambda j: (0, 0)),
            pl.BlockSpec((M, Dr), lambda j: (0, 0)),
        ],
        out_shape=[
            jax.ShapeDtypeStruct((M, D), BF),
            jax.ShapeDtypeStruct((M, H * Dr), BF),
            jax.ShapeDtypeStruct((M, D), BF),
            jax.ShapeDtypeStruct((M, D), BF),
            jax.ShapeDtypeStruct((M, Dr), BF),
        ],
        scratch_shapes=[
            pltpu.VMEM((M, D), BF),
            pltpu.VMEM((D, DC), BF),
            pltpu.VMEM((DC, D), BF),
            pltpu.VMEM((DC, D), BF),
            pltpu.VMEM((D, DC), BF),
            pltpu.VMEM((DC, D), BF),
            pltpu.VMEM((DC, D), BF),
            pltpu.SemaphoreType.DMA((3,)),
            pltpu.SemaphoreType.DMA((3,)),
        ],
        compiler_params=pltpu.CompilerParams(
            collective_id=0, vmem_limit_bytes=60 * 1024 * 1024
        ),
    )(x, Wdkv, Wuk, Wuv, Wkr, Wq, Wqr)

    out = pl.pallas_call(
        _attn_o_body,
        grid=(B,),
        in_specs=[
            pl.BlockSpec((S, D), lambda b: (b, 0)),
            pl.BlockSpec((S, D), lambda b: (b, 0)),
            pl.BlockSpec((S, D), lambda b: (b, 0)),
            pl.BlockSpec((S, H * Dr), lambda b: (b, 0)),
            pl.BlockSpec((S, Dr), lambda b: (b, 0)),
            pl.BlockSpec((D, D), lambda b: (0, 0)),
        ],
        out_specs=pl.BlockSpec((1, S, D), lambda b: (b, 0, 0)),
        out_shape=jax.ShapeDtypeStruct((B, S, D), F32),
        scratch_shapes=[
            pltpu.VMEM((S, D), BF),
            pltpu.VMEM((D, D), BF),
        ],
        compiler_params=pltpu.CompilerParams(
            vmem_limit_bytes=60 * 1024 * 1024
        ),
    )(Q, K, V, Qr, Kr, Wo)
    return out


# device time: 73867 ns/iter; 1.6808x vs baseline; 1.0800x over previous
import jax
import jax.numpy as jnp
from jax import lax
from jax.experimental import pallas as pl
from jax.experimental.pallas import tpu as pltpu

B, S, D = 2, 512, 2048
H, Dh, Dr = 16, 128, 32
DC = 128
M = B * S
SCALE = (Dh + Dr) ** -0.5
BF = jnp.bfloat16
F32 = jnp.float32
NJ = 4


LOG2E = 1.4426950408889634


def _proj_comm_body(x_ref, wdkv_ref, wuk_ref, wuv_ref, wkr_ref, wq_ref, wqr_ref,
                    q_ref, qr_ref, k_ref, v_ref, kr_ref,
                    xb_s, c_s, wuk_s, wuv_s, c_r, wuk_r, wuv_r,
                    send_sems, recv_sems):
    j = pl.program_id(0)
    my_x = lax.axis_index("x")
    my_y = lax.axis_index("y")
    my_z = lax.axis_index("z")
    nbr = (1 - my_x, my_y, my_z)

    pairs = [(c_s, c_r), (wuk_s, wuk_r), (wuv_s, wuv_r)]

    def mk(i, src, dst):
        return pltpu.make_async_remote_copy(
            src_ref=src, dst_ref=dst,
            send_sem=send_sems.at[i], recv_sem=recv_sems.at[i],
            device_id=nbr, device_id_type=pl.DeviceIdType.MESH,
        )

    @pl.when(j == 0)
    def _():
        xb = x_ref[...].reshape(M, D).astype(BF)
        xb_s[...] = xb
        c_s[...] = jnp.dot(
            xb, wdkv_ref[...].astype(BF), preferred_element_type=F32
        ).astype(BF)
        wuk_s[...] = wuk_ref[...].astype(BF)
        wuv_s[...] = wuv_ref[...].astype(BF)
        barrier = pltpu.get_barrier_semaphore()
        pl.semaphore_signal(barrier, inc=1, device_id=nbr,
                            device_id_type=pl.DeviceIdType.MESH)
        pl.semaphore_wait(barrier, 1)
        for i, (s, d) in enumerate(pairs):
            mk(i, s, d).start()

    xb = xb_s[...]
    q_ref[...] = (
        jnp.dot(xb, wq_ref[...].astype(BF), preferred_element_type=F32)
        * (SCALE * LOG2E)
    ).astype(BF)
    qr_ref[...] = (
        jnp.dot(xb, wqr_ref[...].astype(BF), preferred_element_type=F32)
        * (SCALE * LOG2E)
    ).astype(BF)

    @pl.when(j == NJ - 1)
    def _():
        kr_ref[...] = jnp.dot(
            xb, wkr_ref[...].astype(BF), preferred_element_type=F32
        ).astype(BF)
        for i, (s, d) in enumerate(pairs):
            mk(i, s, d).wait()
        k = jnp.dot(c_s[...], wuk_s[...], preferred_element_type=F32)
        k += jnp.dot(c_r[...], wuk_r[...], preferred_element_type=F32)
        k_ref[...] = k.astype(BF)
        v = jnp.dot(c_s[...], wuv_s[...], preferred_element_type=F32)
        v += jnp.dot(c_r[...], wuv_r[...], preferred_element_type=F32)
        v_ref[...] = v.astype(BF)


def _attn_o_body(q_ref, k_ref, v_ref, qr_ref, kr_ref, wo_ref, out_ref,
                 o_s, wo_bf):
    b = pl.program_id(0)

    @pl.when(b == 0)
    def _():
        wo_bf[...] = wo_ref[...].astype(BF)

    nt = (((1,), (1,)), ((), ()))
    kr = kr_ref[...]
    for h in range(H):
        q = q_ref[:, h * Dh:(h + 1) * Dh]
        k = k_ref[:, h * Dh:(h + 1) * Dh]
        qr = qr_ref[:, h * Dr:(h + 1) * Dr]
        s = lax.dot_general(q, k, nt, preferred_element_type=F32)
        s += lax.dot_general(qr, kr, nt, preferred_element_type=F32)
        p = jnp.exp2(s)
        r = 1.0 / jnp.sum(p, axis=-1, keepdims=True)
        o = jnp.dot(
            p.astype(BF), v_ref[:, h * Dh:(h + 1) * Dh],
            preferred_element_type=F32,
        )
        o_s[:, h * Dh:(h + 1) * Dh] = (o * r).astype(BF)

    out = jnp.dot(o_s[...], wo_bf[...], preferred_element_type=F32)
    out_ref[...] = out.reshape(1, S, D)


def kernel(x, Wdkv, Wuk, Wuv, Wq, Wqr, Wkr, Wo):
    BD = D // NJ
    BR = H * Dr // NJ

    Q, Qr, K, V, Kr = pl.pallas_call(
        _proj_comm_body,
        grid=(NJ,),
        in_specs=[
            pl.BlockSpec((B, S, D), lambda j: (0, 0, 0)),
            pl.BlockSpec((D, DC), lambda j: (0, 0)),
            pl.BlockSpec((DC, D), lambda j: (0, 0)),
            pl.BlockSpec((DC, D), lambda j: (0, 0)),
            pl.BlockSpec((D, Dr), lambda j: (0, 0)),
            pl.BlockSpec((D, BD), lambda j: (0, j)),
            pl.BlockSpec((D, BR), lambda j: (0, j)),
        ],
        out_specs=[
            pl.BlockSpec((M, BD), lambda j: (0, j)),
            pl.BlockSpec((M, BR), lambda j: (0, j)),
            pl.BlockSpec((M, D), lambda j: (0, 0)),
            pl.BlockSpec((M, D), lambda j: (0, 0)),
            pl.BlockSpec((M, Dr), lambda j: (0, 0)),
        ],
        out_shape=[
            jax.ShapeDtypeStruct((M, D), BF),
            jax.ShapeDtypeStruct((M, H * Dr), BF),
            jax.ShapeDtypeStruct((M, D), BF),
            jax.ShapeDtypeStruct((M, D), BF),
            jax.ShapeDtypeStruct((M, Dr), BF),
        ],
        scratch_shapes=[
            pltpu.VMEM((M, D), BF),
            pltpu.VMEM((M, DC), BF),
            pltpu.VMEM((DC, D), BF),
            pltpu.VMEM((DC, D), BF),
            pltpu.VMEM((M, DC), BF),
            pltpu.VMEM((DC, D), BF),
            pltpu.VMEM((DC, D), BF),
            pltpu.SemaphoreType.DMA((3,)),
            pltpu.SemaphoreType.DMA((3,)),
        ],
        compiler_params=pltpu.CompilerParams(
            collective_id=0, vmem_limit_bytes=60 * 1024 * 1024
        ),
    )(x, Wdkv, Wuk, Wuv, Wkr, Wq, Wqr)

    out = pl.pallas_call(
        _attn_o_body,
        grid=(B,),
        in_specs=[
            pl.BlockSpec((S, D), lambda b: (b, 0)),
            pl.BlockSpec((S, D), lambda b: (b, 0)),
            pl.BlockSpec((S, D), lambda b: (b, 0)),
            pl.BlockSpec((S, H * Dr), lambda b: (b, 0)),
            pl.BlockSpec((S, Dr), lambda b: (b, 0)),
            pl.BlockSpec((D, D), lambda b: (0, 0)),
        ],
        out_specs=pl.BlockSpec((1, S, D), lambda b: (b, 0, 0)),
        out_shape=jax.ShapeDtypeStruct((B, S, D), F32),
        scratch_shapes=[
            pltpu.VMEM((S, D), BF),
            pltpu.VMEM((D, D), BF),
        ],
        compiler_params=pltpu.CompilerParams(
            vmem_limit_bytes=60 * 1024 * 1024
        ),
    )(Q, K, V, Qr, Kr, Wo)
    return out


# device time: 72185 ns/iter; 1.7200x vs baseline; 1.0233x over previous
import jax
import jax.numpy as jnp
from jax import lax
from jax.experimental import pallas as pl
from jax.experimental.pallas import tpu as pltpu

B, S, D = 2, 512, 2048
H, Dh, Dr = 16, 128, 32
DC = 128
M = B * S
SCALE = (Dh + Dr) ** -0.5
BF = jnp.bfloat16
F32 = jnp.float32
NJ = 4


LOG2E = 1.4426950408889634


def _proj_comm_body(x_ref, wdkv_ref, wuk_ref, wuv_ref, wkr_ref, wq_ref, wqr_ref,
                    q_ref, qr_ref, k_ref, v_ref, kr_ref,
                    xb_s, c_s, wuk_s, wuv_s, c_r, wuk_r, wuv_r,
                    send_sems, recv_sems):
    j = pl.program_id(0)
    my_x = lax.axis_index("x")
    my_y = lax.axis_index("y")
    my_z = lax.axis_index("z")
    nbr = (1 - my_x, my_y, my_z)

    pairs = [(c_s, c_r), (wuk_s, wuk_r), (wuv_s, wuv_r)]

    def mk(i, src, dst):
        return pltpu.make_async_remote_copy(
            src_ref=src, dst_ref=dst,
            send_sem=send_sems.at[i], recv_sem=recv_sems.at[i],
            device_id=nbr, device_id_type=pl.DeviceIdType.MESH,
        )

    @pl.when(j == 0)
    def _():
        barrier = pltpu.get_barrier_semaphore()
        pl.semaphore_signal(barrier, inc=1, device_id=nbr,
                            device_id_type=pl.DeviceIdType.MESH)
        xb = x_ref[...].reshape(M, D).astype(BF)
        xb_s[...] = xb
        c_s[...] = jnp.dot(
            xb, wdkv_ref[...].astype(BF), preferred_element_type=F32
        ).astype(BF)
        wuk_s[...] = wuk_ref[...].astype(BF)
        wuv_s[...] = wuv_ref[...].astype(BF)
        pl.semaphore_wait(barrier, 1)
        for i, (s, d) in enumerate(pairs):
            mk(i, s, d).start()

    xb = xb_s[...]
    q_ref[...] = (
        jnp.dot(xb, wq_ref[...].astype(BF), preferred_element_type=F32)
        * (SCALE * LOG2E)
    ).astype(BF)
    qr_ref[...] = (
        jnp.dot(xb, wqr_ref[...].astype(BF), preferred_element_type=F32)
        * (SCALE * LOG2E)
    ).astype(BF)

    @pl.when(j == NJ - 1)
    def _():
        kr_ref[...] = jnp.dot(
            xb, wkr_ref[...].astype(BF), preferred_element_type=F32
        ).astype(BF)
        for i, (s, d) in enumerate(pairs):
            mk(i, s, d).wait()
        k = jnp.dot(c_s[...], wuk_s[...], preferred_element_type=F32)
        k += jnp.dot(c_r[...], wuk_r[...], preferred_element_type=F32)
        k_ref[...] = k.astype(BF)
        v = jnp.dot(c_s[...], wuv_s[...], preferred_element_type=F32)
        v += jnp.dot(c_r[...], wuv_r[...], preferred_element_type=F32)
        v_ref[...] = v.astype(BF)


def _attn_o_body(q_ref, k_ref, v_ref, qr_ref, kr_ref, wo_ref, out_ref,
                 o_s, wo_bf):
    b = pl.program_id(0)

    @pl.when(b == 0)
    def _():
        wo_bf[...] = wo_ref[...].astype(BF)

    nt = (((1,), (1,)), ((), ()))
    kr = kr_ref[...]
    for h in range(H):
        q = q_ref[:, h * Dh:(h + 1) * Dh]
        k = k_ref[:, h * Dh:(h + 1) * Dh]
        qr = qr_ref[:, h * Dr:(h + 1) * Dr]
        s = lax.dot_general(q, k, nt, preferred_element_type=F32)
        s += lax.dot_general(qr, kr, nt, preferred_element_type=F32)
        p = jnp.exp2(s.astype(BF))
        r = 1.0 / jnp.sum(p.astype(F32), axis=-1, keepdims=True)
        o = jnp.dot(
            p, v_ref[:, h * Dh:(h + 1) * Dh],
            preferred_element_type=F32,
        )
        o_s[:, h * Dh:(h + 1) * Dh] = (o * r).astype(BF)

    out = jnp.dot(o_s[...], wo_bf[...], preferred_element_type=F32)
    out_ref[...] = out.reshape(1, S, D)


def kernel(x, Wdkv, Wuk, Wuv, Wq, Wqr, Wkr, Wo):
    BD = D // NJ
    BR = H * Dr // NJ

    Q, Qr, K, V, Kr = pl.pallas_call(
        _proj_comm_body,
        grid=(NJ,),
        in_specs=[
            pl.BlockSpec((B, S, D), lambda j: (0, 0, 0)),
            pl.BlockSpec((D, DC), lambda j: (0, 0)),
            pl.BlockSpec((DC, D), lambda j: (0, 0)),
            pl.BlockSpec((DC, D), lambda j: (0, 0)),
            pl.BlockSpec((D, Dr), lambda j: (0, 0)),
            pl.BlockSpec((D, BD), lambda j: (0, j)),
            pl.BlockSpec((D, BR), lambda j: (0, j)),
        ],
        out_specs=[
            pl.BlockSpec((M, BD), lambda j: (0, j)),
            pl.BlockSpec((M, BR), lambda j: (0, j)),
            pl.BlockSpec((M, D), lambda j: (0, 0)),
            pl.BlockSpec((M, D), lambda j: (0, 0)),
            pl.BlockSpec((M, Dr), lambda j: (0, 0)),
        ],
        out_shape=[
            jax.ShapeDtypeStruct((M, D), BF),
            jax.ShapeDtypeStruct((M, H * Dr), BF),
            jax.ShapeDtypeStruct((M, D), BF),
            jax.ShapeDtypeStruct((M, D), BF),
            jax.ShapeDtypeStruct((M, Dr), BF),
        ],
        scratch_shapes=[
            pltpu.VMEM((M, D), BF),
            pltpu.VMEM((M, DC), BF),
            pltpu.VMEM((DC, D), BF),
            pltpu.VMEM((DC, D), BF),
            pltpu.VMEM((M, DC), BF),
            pltpu.VMEM((DC, D), BF),
            pltpu.VMEM((DC, D), BF),
            pltpu.SemaphoreType.DMA((3,)),
            pltpu.SemaphoreType.DMA((3,)),
        ],
        compiler_params=pltpu.CompilerParams(
            collective_id=0, vmem_limit_bytes=60 * 1024 * 1024
        ),
    )(x, Wdkv, Wuk, Wuv, Wkr, Wq, Wqr)

    out = pl.pallas_call(
        _attn_o_body,
        grid=(B,),
        in_specs=[
            pl.BlockSpec((S, D), lambda b: (b, 0)),
            pl.BlockSpec((S, D), lambda b: (b, 0)),
            pl.BlockSpec((S, D), lambda b: (b, 0)),
            pl.BlockSpec((S, H * Dr), lambda b: (b, 0)),
            pl.BlockSpec((S, Dr), lambda b: (b, 0)),
            pl.BlockSpec((D, D), lambda b: (0, 0)),
        ],
        out_specs=pl.BlockSpec((1, S, D), lambda b: (b, 0, 0)),
        out_shape=jax.ShapeDtypeStruct((B, S, D), F32),
        scratch_shapes=[
            pltpu.VMEM((S, D), BF),
            pltpu.VMEM((D, D), BF),
        ],
        compiler_params=pltpu.CompilerParams(
            vmem_limit_bytes=60 * 1024 * 1024
        ),
    )(Q, K, V, Qr, Kr, Wo)
    return out


# device time: 68595 ns/iter; 1.8100x vs baseline; 1.0523x over previous
import jax
import jax.numpy as jnp
from jax import lax
from jax.experimental import pallas as pl
from jax.experimental.pallas import tpu as pltpu

B, S, D = 2, 512, 2048
H, Dh, Dr = 16, 128, 32
DC = 128
M = B * S
SCALE = (Dh + Dr) ** -0.5
BF = jnp.bfloat16
F32 = jnp.float32
NJ = 4


LOG2E = 1.4426950408889634


def _proj_comm_body(x_ref, wdkv_ref, wuk_ref, wuv_ref, wkr_ref, wq_ref, wqr_ref,
                    q_ref, k_ref, v_ref, kr_ref,
                    xb_s, c_s, wuk_s, wuv_s, c_r, wuk_r, wuv_r,
                    send_sems, recv_sems):
    j = pl.program_id(0)
    my_x = lax.axis_index("x")
    my_y = lax.axis_index("y")
    my_z = lax.axis_index("z")
    nbr = (1 - my_x, my_y, my_z)

    pairs = [(c_s, c_r), (wuk_s, wuk_r), (wuv_s, wuv_r)]

    def mk(i, src, dst):
        return pltpu.make_async_remote_copy(
            src_ref=src, dst_ref=dst,
            send_sem=send_sems.at[i], recv_sem=recv_sems.at[i],
            device_id=nbr, device_id_type=pl.DeviceIdType.MESH,
        )

    @pl.when(j == 0)
    def _():
        barrier = pltpu.get_barrier_semaphore()
        pl.semaphore_signal(barrier, inc=1, device_id=nbr,
                            device_id_type=pl.DeviceIdType.MESH)
        xb = x_ref[...].reshape(M, D).astype(BF)
        xb_s[...] = xb
        c_s[...] = jnp.dot(
            xb, wdkv_ref[...].astype(BF), preferred_element_type=F32
        ).astype(BF)
        wuk_s[...] = wuk_ref[...].astype(BF)
        wuv_s[...] = wuv_ref[...].astype(BF)
        pl.semaphore_wait(barrier, 1)
        for i, (s, d) in enumerate(pairs):
            mk(i, s, d).start()

    xb = xb_s[...]
    qd = (
        jnp.dot(xb, wq_ref[...].astype(BF), preferred_element_type=F32)
        * (SCALE * LOG2E)
    ).astype(BF)
    qrd = (
        jnp.dot(xb, wqr_ref[...].astype(BF), preferred_element_type=F32)
        * (SCALE * LOG2E)
    ).astype(BF)
    zpad = jnp.zeros((M, Dh - Dr), BF)
    for i in range(4):
        q_ref[:, i * 256:i * 256 + Dh] = qd[:, i * Dh:(i + 1) * Dh]
        q_ref[:, i * 256 + Dh:(i + 1) * 256] = jnp.concatenate(
            [qrd[:, i * Dr:(i + 1) * Dr], zpad], axis=1
        )

    @pl.when(j == NJ - 1)
    def _():
        kr_ref[...] = jnp.dot(
            xb, wkr_ref[...].astype(BF), preferred_element_type=F32
        ).astype(BF)
        for i, (s, d) in enumerate(pairs):
            mk(i, s, d).wait()
        k = jnp.dot(c_s[...], wuk_s[...], preferred_element_type=F32)
        k += jnp.dot(c_r[...], wuk_r[...], preferred_element_type=F32)
        k_ref[...] = k.astype(BF)
        v = jnp.dot(c_s[...], wuv_s[...], preferred_element_type=F32)
        v += jnp.dot(c_r[...], wuv_r[...], preferred_element_type=F32)
        v_ref[...] = v.astype(BF)


def _attn_o_body(q_ref, k_ref, v_ref, kr_ref, wo_ref, out_ref,
                 o_s, wo_bf):
    b = pl.program_id(0)

    @pl.when(b == 0)
    def _():
        wo_bf[...] = wo_ref[...].astype(BF)

    nt = (((1,), (1,)), ((), ()))
    kr = kr_ref[...]
    for h in range(H):
        q = q_ref[:, h * 256:(h + 1) * 256]
        k = k_ref[:, h * Dh:(h + 1) * Dh]
        kcat = jnp.concatenate([k, kr, k[:, :Dh - Dr]], axis=1)
        s = lax.dot_general(q, kcat, nt, preferred_element_type=F32)
        p = jnp.exp2(s.astype(BF))
        r = 1.0 / jnp.sum(p.astype(F32), axis=-1, keepdims=True)
        o = jnp.dot(
            p, v_ref[:, h * Dh:(h + 1) * Dh],
            preferred_element_type=F32,
        )
        o_s[:, h * Dh:(h + 1) * Dh] = (o * r).astype(BF)

    out = jnp.dot(o_s[...], wo_bf[...], preferred_element_type=F32)
    out_ref[...] = out.reshape(1, S, D)


def kernel(x, Wdkv, Wuk, Wuv, Wq, Wqr, Wkr, Wo):
    BD = D // NJ
    BR = H * Dr // NJ

    Q, K, V, Kr = pl.pallas_call(
        _proj_comm_body,
        grid=(NJ,),
        in_specs=[
            pl.BlockSpec((B, S, D), lambda j: (0, 0, 0)),
            pl.BlockSpec((D, DC), lambda j: (0, 0)),
            pl.BlockSpec((DC, D), lambda j: (0, 0)),
            pl.BlockSpec((DC, D), lambda j: (0, 0)),
            pl.BlockSpec((D, Dr), lambda j: (0, 0)),
            pl.BlockSpec((D, BD), lambda j: (0, j)),
            pl.BlockSpec((D, BR), lambda j: (0, j)),
        ],
        out_specs=[
            pl.BlockSpec((M, 4 * 256), lambda j: (0, j)),
            pl.BlockSpec((M, D), lambda j: (0, 0)),
            pl.BlockSpec((M, D), lambda j: (0, 0)),
            pl.BlockSpec((M, Dr), lambda j: (0, 0)),
        ],
        out_shape=[
            jax.ShapeDtypeStruct((M, H * 256), BF),
            jax.ShapeDtypeStruct((M, D), BF),
            jax.ShapeDtypeStruct((M, D), BF),
            jax.ShapeDtypeStruct((M, Dr), BF),
        ],
        scratch_shapes=[
            pltpu.VMEM((M, D), BF),
            pltpu.VMEM((M, DC), BF),
            pltpu.VMEM((DC, D), BF),
            pltpu.VMEM((DC, D), BF),
            pltpu.VMEM((M, DC), BF),
            pltpu.VMEM((DC, D), BF),
            pltpu.VMEM((DC, D), BF),
            pltpu.SemaphoreType.DMA((3,)),
            pltpu.SemaphoreType.DMA((3,)),
        ],
        compiler_params=pltpu.CompilerParams(
            collective_id=0, vmem_limit_bytes=60 * 1024 * 1024
        ),
    )(x, Wdkv, Wuk, Wuv, Wkr, Wq, Wqr)

    out = pl.pallas_call(
        _attn_o_body,
        grid=(B,),
        in_specs=[
            pl.BlockSpec((S, H * 256), lambda b: (b, 0)),
            pl.BlockSpec((S, D), lambda b: (b, 0)),
            pl.BlockSpec((S, D), lambda b: (b, 0)),
            pl.BlockSpec((S, Dr), lambda b: (b, 0)),
            pl.BlockSpec((D, D), lambda b: (0, 0)),
        ],
        out_specs=pl.BlockSpec((1, S, D), lambda b: (b, 0, 0)),
        out_shape=jax.ShapeDtypeStruct((B, S, D), F32),
        scratch_shapes=[
            pltpu.VMEM((S, D), BF),
            pltpu.VMEM((D, D), BF),
        ],
        compiler_params=pltpu.CompilerParams(
            vmem_limit_bytes=60 * 1024 * 1024
        ),
    )(Q, K, V, Kr, Wo)
    return out
